# baseline (speedup 1.0000x reference)
import functools

import jax
import jax.numpy as jnp
from jax import lax
from jax.experimental import pallas as pl
from jax.experimental.pallas import tpu as pltpu
from jax.experimental.pallas import tpu_sc as plsc

L = 16
NC, NS = 2, 16
NW = NC * NS


@functools.cache
def _build(B, C, HW, S):
    RPW = (B * C) // NW
    CCH = min(64, RPW)
    G = min(4, CCH)
    NG = CCH // G
    NCHUNK = RPW // CCH
    PV = HW // L
    mesh = plsc.VectorSubcoreMesh(core_axis_name="c", subcore_axis_name="s")

    @functools.partial(
        pl.kernel,
        mesh=mesh,
        out_type=jax.ShapeDtypeStruct((B * C, HW), jnp.float32),
        scratch_types=[
            pltpu.VMEM((HW,), jnp.int32),
            pltpu.VMEM((S, CCH), jnp.float32),
            pltpu.VMEM((S, CCH), jnp.float32),
            pltpu.VMEM((G, HW), jnp.float32),
            pltpu.VMEM((G, HW), jnp.float32),
            pltpu.VMEM((G, HW), jnp.float32),
            pltpu.VMEM((G, HW), jnp.float32),
            pltpu.SemaphoreType.DMA,
            pltpu.SemaphoreType.DMA,
            pltpu.SemaphoreType.DMA,
            pltpu.SemaphoreType.DMA,
        ],
        compiler_params=pltpu.CompilerParams(
            needs_layout_passes=False, use_tc_tiling_on_sc=False),
    )
    def _slotfa(x_hbm, slot_hbm, a_hbm, b_hbm, out_hbm,
                slot_v, at_v, bt_v, xb0, xb1, yb0, yb1,
                sx0, sx1, sy0, sy1):
        xb, yb, sx, sy = (xb0, xb1), (yb0, yb1), (sx0, sx1), (sy0, sy1)
        wid = lax.axis_index("s") * NC + lax.axis_index("c")
        r0 = wid * RPW
        bidx = r0 // C
        c0 = r0 % C
        pltpu.sync_copy(slot_hbm.at[bidx], slot_v)
        for cc in range(NCHUNK):
            cbase = c0 + cc * CCH
            rbase = r0 + cc * CCH
            pltpu.sync_copy(a_hbm.at[:, pl.ds(cbase, CCH)], at_v)
            pltpu.sync_copy(b_hbm.at[:, pl.ds(cbase, CCH)], bt_v)
            pltpu.async_copy(x_hbm.at[pl.ds(rbase, G)], xb0, sx0)
            pltpu.async_copy(x_hbm.at[pl.ds(rbase + G, G)], xb1, sx1)

            def gp_body(gp, _, rbase=rbase):
                for par in range(2):
                    g = gp * 2 + par
                    rg = rbase + g * G
                    pltpu.make_async_copy(
                        x_hbm.at[pl.ds(rg, G)], xb[par], sx[par]).wait()

                    @pl.when(gp >= 1)
                    def _():
                        pltpu.make_async_copy(
                            yb[par], out_hbm.at[pl.ds(rg, G)], sy[par]).wait()

                    cv = [jnp.full((L,), g * G + j, jnp.int32)
                          for j in range(G)]

                    @plsc.parallel_loop(0, PV, 1, unroll=8)
                    def pv_body(pv):
                        o = pv * L
                        sl = slot_v[pl.ds(o, L)]
                        for j in range(G):
                            av = plsc.load_gather(at_v, [sl, cv[j]])
                            bv = plsc.load_gather(bt_v, [sl, cv[j]])
                            yb[par][j, pl.ds(o, L)] = (
                                av * xb[par][j, pl.ds(o, L)] + bv)

                    pltpu.async_copy(yb[par], out_hbm.at[pl.ds(rg, G)],
                                     sy[par])

                    @pl.when(g + 2 < NG)
                    def _():
                        pltpu.async_copy(
                            x_hbm.at[pl.ds(rg + 2 * G, G)], xb[par], sx[par])

                return 0

            lax.fori_loop(0, NG // 2, gp_body, 0)
            pltpu.make_async_copy(
                yb0, out_hbm.at[pl.ds(rbase + (NG - 2) * G, G)], sy0).wait()
            pltpu.make_async_copy(
                yb1, out_hbm.at[pl.ds(rbase + (NG - 1) * G, G)], sy1).wait()

    return _slotfa


@jax.jit
def kernel(x, slot_assign, alphas, betas):
    b, c, h, w = x.shape
    s = alphas.shape[0]
    x2 = x.reshape(b * c, h * w)
    slot2 = slot_assign.reshape(b, h * w).astype(jnp.int32)
    out2 = _build(b, c, h * w, s)(x2, slot2, alphas, betas)
    return out2.reshape(x.shape)


# speedup vs baseline: 2.8092x; 2.8092x over previous
import functools

import jax
import jax.numpy as jnp
from jax import lax
from jax.experimental import pallas as pl
from jax.experimental.pallas import tpu as pltpu
from jax.experimental.pallas import tpu_sc as plsc

L = 16
NC, NS = 2, 16
NW = NC * NS


@functools.cache
def _build(B, C, H, W, S):
    CPW = (B * C) // NW
    CCH = min(64, CPW)
    G = min(2, CCH)
    NG = CCH // G
    NCHUNK = CPW // CCH
    PV = (H * W) // L
    WV = W // L
    mesh = plsc.VectorSubcoreMesh(core_axis_name="c", subcore_axis_name="s")

    @functools.partial(
        pl.kernel,
        mesh=mesh,
        out_type=jax.ShapeDtypeStruct((B, C, H, W), jnp.float32),
        scratch_types=[
            pltpu.VMEM((H, W), jnp.int32),
            pltpu.VMEM((CCH * S,), jnp.float32),
            pltpu.VMEM((CCH * S,), jnp.float32),
            pltpu.VMEM((G, H, W), jnp.float32),
            pltpu.VMEM((G, H, W), jnp.float32),
            pltpu.VMEM((G, H, W), jnp.float32),
            pltpu.VMEM((G, H, W), jnp.float32),
            pltpu.SemaphoreType.DMA,
            pltpu.SemaphoreType.DMA,
            pltpu.SemaphoreType.DMA,
            pltpu.SemaphoreType.DMA,
        ],
        compiler_params=pltpu.CompilerParams(needs_layout_passes=False),
    )
    def _slotfa(x_hbm, slot_hbm, a_hbm, b_hbm, out_hbm,
                slot_v, at_v, bt_v, xb0, xb1, yb0, yb1,
                sx0, sx1, sy0, sy1):
        xb, yb, sx, sy = (xb0, xb1), (yb0, yb1), (sx0, sx1), (sy0, sy1)
        wid = lax.axis_index("s") * NC + lax.axis_index("c")
        p0 = wid * CPW
        bidx = p0 // C
        c0 = p0 % C
        pltpu.sync_copy(slot_hbm.at[bidx], slot_v)
        for cc in range(NCHUNK):
            cbase = c0 + cc * CCH
            pltpu.sync_copy(a_hbm.at[pl.ds(cbase * S, CCH * S)], at_v)
            pltpu.sync_copy(b_hbm.at[pl.ds(cbase * S, CCH * S)], bt_v)
            pltpu.async_copy(x_hbm.at[bidx, pl.ds(cbase, G)], xb0, sx0)
            pltpu.async_copy(x_hbm.at[bidx, pl.ds(cbase + G, G)], xb1, sx1)

            def gp_body(gp, _, cbase=cbase, bidx=bidx):
                for par in range(2):
                    g = gp * 2 + par
                    cg = cbase + g * G
                    pltpu.make_async_copy(
                        x_hbm.at[bidx, pl.ds(cg, G)], xb[par], sx[par]).wait()

                    @pl.when(gp >= 1)
                    def _():
                        pltpu.make_async_copy(
                            yb[par], out_hbm.at[bidx, pl.ds(cg, G)],
                            sy[par]).wait()

                    cv = [jnp.full((L,), (g * G + j) * S, jnp.int32)
                          for j in range(G)]

                    @plsc.parallel_loop(0, PV, 1, unroll=8)
                    def pv_body(pv):
                        hh = pv // WV
                        wq = (pv % WV) * L
                        sl = slot_v[hh, pl.ds(wq, L)]
                        for j in range(G):
                            idx = sl + cv[j]
                            av = plsc.load_gather(at_v, [idx])
                            bv = plsc.load_gather(bt_v, [idx])
                            yb[par][j, hh, pl.ds(wq, L)] = (
                                av * xb[par][j, hh, pl.ds(wq, L)] + bv)

                    pltpu.async_copy(yb[par], out_hbm.at[bidx, pl.ds(cg, G)],
                                     sy[par])

                    @pl.when(g + 2 < NG)
                    def _():
                        pltpu.async_copy(
                            x_hbm.at[bidx, pl.ds(cg + 2 * G, G)],
                            xb[par], sx[par])

                return 0

            lax.fori_loop(0, NG // 2, gp_body, 0)
            pltpu.make_async_copy(
                yb0, out_hbm.at[bidx, pl.ds(cbase + (NG - 2) * G, G)],
                sy0).wait()
            pltpu.make_async_copy(
                yb1, out_hbm.at[bidx, pl.ds(cbase + (NG - 1) * G, G)],
                sy1).wait()

    return _slotfa


@jax.jit
def kernel(x, slot_assign, alphas, betas):
    b, c, h, w = x.shape
    s = alphas.shape[0]
    slot3 = slot_assign.astype(jnp.int32)
    a_t = alphas.T.reshape(-1)
    b_t = betas.T.reshape(-1)
    return _build(b, c, h, w, s)(x, slot3, a_t, b_t)


# speedup vs baseline: 4.0640x; 1.4466x over previous
import functools

import jax
import jax.numpy as jnp
from jax import lax
from jax.experimental import pallas as pl
from jax.experimental.pallas import tpu as pltpu
from jax.experimental.pallas import tpu_sc as plsc

L = 16
NC, NS = 2, 16
NW = NC * NS


@functools.cache
def _build(P, C, S):
    NCR = 16
    NPQ = NW // NCR
    CCH = C // NCR
    PPW = P // NPQ
    GP = 64
    NGRP = PPW // GP
    CV = CCH // L
    QB = GP // L
    mesh = plsc.VectorSubcoreMesh(core_axis_name="c", subcore_axis_name="s")

    @functools.partial(
        pl.kernel,
        mesh=mesh,
        out_type=jax.ShapeDtypeStruct((P, C), jnp.float32),
        scratch_types=[
            pltpu.VMEM((PPW,), jnp.int32),
            pltpu.VMEM((S, CCH), jnp.float32),
            pltpu.VMEM((S, CCH), jnp.float32),
            pltpu.VMEM((GP, CCH), jnp.float32),
            pltpu.VMEM((GP, CCH), jnp.float32),
            pltpu.VMEM((GP, CCH), jnp.float32),
            pltpu.VMEM((GP, CCH), jnp.float32),
            pltpu.SemaphoreType.DMA,
            pltpu.SemaphoreType.DMA,
            pltpu.SemaphoreType.DMA,
            pltpu.SemaphoreType.DMA,
        ],
        compiler_params=pltpu.CompilerParams(needs_layout_passes=False),
    )
    def _slotfa(x_hbm, slot_hbm, a_hbm, b_hbm, out_hbm,
                slot_v, at_v, bt_v, xb0, xb1, yb0, yb1,
                sx0, sx1, sy0, sy1):
        xb, yb, sx, sy = (xb0, xb1), (yb0, yb1), (sx0, sx1), (sy0, sy1)
        wid = lax.axis_index("s") * NC + lax.axis_index("c")
        cr = wid % NCR
        pq = wid // NCR
        c0 = cr * CCH
        p0 = pq * PPW
        pltpu.sync_copy(slot_hbm.at[pl.ds(p0, PPW)], slot_v)
        pltpu.sync_copy(a_hbm.at[:, pl.ds(c0, CCH)], at_v)
        pltpu.sync_copy(b_hbm.at[:, pl.ds(c0, CCH)], bt_v)
        pltpu.async_copy(
            x_hbm.at[pl.ds(p0, GP), pl.ds(c0, CCH)], xb0, sx0)
        pltpu.async_copy(
            x_hbm.at[pl.ds(p0 + GP, GP), pl.ds(c0, CCH)], xb1, sx1)

        def gp_body(gp, _):
            for par in range(2):
                g = gp * 2 + par
                pg = p0 + g * GP
                pltpu.make_async_copy(
                    x_hbm.at[pl.ds(pg, GP), pl.ds(c0, CCH)],
                    xb[par], sx[par]).wait()

                @pl.when(gp >= 1)
                def _():
                    pltpu.make_async_copy(
                        yb[par], out_hbm.at[pl.ds(pg, GP), pl.ds(c0, CCH)],
                        sy[par]).wait()

                @plsc.parallel_loop(0, QB, 1)
                def q_body(q, g=g):
                    sl = slot_v[pl.ds(g * GP + q * L, L)]
                    for j in range(L):
                        s = sl[j]
                        r = q * L + j
                        for v in range(CV):
                            o = v * L
                            yb[par][r, pl.ds(o, L)] = (
                                at_v[s, pl.ds(o, L)]
                                * xb[par][r, pl.ds(o, L)]
                                + bt_v[s, pl.ds(o, L)])

                pltpu.async_copy(
                    yb[par], out_hbm.at[pl.ds(pg, GP), pl.ds(c0, CCH)],
                    sy[par])

                @pl.when(g + 2 < NGRP)
                def _():
                    pltpu.async_copy(
                        x_hbm.at[pl.ds(pg + 2 * GP, GP), pl.ds(c0, CCH)],
                        xb[par], sx[par])

            return 0

        lax.fori_loop(0, NGRP // 2, gp_body, 0)
        pltpu.make_async_copy(
            yb0, out_hbm.at[pl.ds(p0 + (NGRP - 2) * GP, GP),
                            pl.ds(c0, CCH)], sy0).wait()
        pltpu.make_async_copy(
            yb1, out_hbm.at[pl.ds(p0 + (NGRP - 1) * GP, GP),
                            pl.ds(c0, CCH)], sy1).wait()

    return _slotfa


@jax.jit
def kernel(x, slot_assign, alphas, betas):
    b, c, h, w = x.shape
    s = alphas.shape[0]
    xt = jnp.transpose(x, (0, 2, 3, 1)).reshape(b * h * w, c)
    slot1 = slot_assign.reshape(b * h * w).astype(jnp.int32)
    out2 = _build(b * h * w, c, s)(xt, slot1, alphas, betas)
    out = jnp.transpose(out2.reshape(b, h, w, c), (0, 3, 1, 2))
    return out


# speedup vs baseline: 8.3274x; 2.0491x over previous
import functools

import jax
import jax.numpy as jnp
from jax import lax
from jax.experimental import pallas as pl
from jax.experimental.pallas import tpu as pltpu
from jax.experimental.pallas import tpu_sc as plsc

L = 16
NC, NS = 2, 16
NW = NC * NS


@functools.cache
def _build(P, C, S):
    NCR = 16
    NPQ = NW // NCR
    CCH = C // NCR
    PPW = P // NPQ
    GP = 32
    NGRP = PPW // GP
    CV = CCH // L
    mesh = plsc.VectorSubcoreMesh(core_axis_name="c", subcore_axis_name="s")

    @functools.partial(
        pl.kernel,
        mesh=mesh,
        out_type=jax.ShapeDtypeStruct((P, C), jnp.float32),
        scratch_types=[
            pltpu.VMEM((PPW,), jnp.int32),
            pltpu.VMEM((PPW,), jnp.int32),
            pltpu.VMEM_SHARED((S * NCR, CCH), jnp.float32),
            pltpu.VMEM_SHARED((S * NCR, CCH), jnp.float32),
            pltpu.VMEM((GP, CCH), jnp.float32),
            pltpu.VMEM((GP, CCH), jnp.float32),
            pltpu.VMEM((GP, CCH), jnp.float32),
            pltpu.VMEM((GP, CCH), jnp.float32),
            pltpu.VMEM((GP, CCH), jnp.float32),
            pltpu.VMEM((GP, CCH), jnp.float32),
            pltpu.VMEM((GP, CCH), jnp.float32),
            pltpu.VMEM((GP, CCH), jnp.float32),
            pltpu.SemaphoreType.DMA,
            pltpu.SemaphoreType.DMA,
            pltpu.SemaphoreType.DMA,
            pltpu.SemaphoreType.DMA,
            pltpu.SemaphoreType.DMA,
            pltpu.SemaphoreType.DMA,
            pltpu.SemaphoreType.DMA,
            pltpu.SemaphoreType.DMA,
        ],
        compiler_params=pltpu.CompilerParams(needs_layout_passes=False),
    )
    def _slotfa(x_hbm, slot_hbm, a_hbm, b_hbm, out_hbm,
                slot_v, idx_v, a_sh, b_sh,
                xb0, xb1, yb0, yb1, ab0, ab1, bb0, bb1,
                sx0, sx1, sy0, sy1, sa0, sa1, sb0, sb1):
        xb, yb, ab, bb = (xb0, xb1), (yb0, yb1), (ab0, ab1), (bb0, bb1)
        sx, sy, sa, sb = (sx0, sx1), (sy0, sy1), (sa0, sa1), (sb0, sb1)
        cidx = lax.axis_index("c")
        sidx = lax.axis_index("s")
        wid = sidx * NC + cidx
        cr = wid % NCR
        pq = wid // NCR
        c0 = cr * CCH
        p0 = pq * PPW

        @pl.when(sidx == 0)
        def _():
            pltpu.sync_copy(a_hbm, a_sh)
            pltpu.sync_copy(b_hbm, b_sh)

        pltpu.sync_copy(slot_hbm.at[pl.ds(p0, PPW)], slot_v)

        @plsc.parallel_loop(0, PPW // L, 1, unroll=8)
        def idx_body(i):
            o = i * L
            idx_v[pl.ds(o, L)] = slot_v[pl.ds(o, L)] * NCR + cr

        plsc.subcore_barrier()

        for par in range(2):
            pltpu.async_copy(
                x_hbm.at[pl.ds(p0 + par * GP, GP), pl.ds(c0, CCH)],
                xb[par], sx[par])
            pltpu.async_copy(
                a_sh.at[idx_v.at[pl.ds(par * GP, GP)]], ab[par], sa[par])
            pltpu.async_copy(
                b_sh.at[idx_v.at[pl.ds(par * GP, GP)]], bb[par], sb[par])

        def gp_body(gp, _):
            for par in range(2):
                g = gp * 2 + par
                pg = p0 + g * GP
                pltpu.make_async_copy(
                    x_hbm.at[pl.ds(pg, GP), pl.ds(c0, CCH)],
                    xb[par], sx[par]).wait()
                pltpu.make_async_copy(
                    a_sh.at[idx_v.at[pl.ds(g * GP, GP)]], ab[par],
                    sa[par]).wait()
                pltpu.make_async_copy(
                    b_sh.at[idx_v.at[pl.ds(g * GP, GP)]], bb[par],
                    sb[par]).wait()

                @pl.when(gp >= 1)
                def _():
                    pltpu.make_async_copy(
                        yb[par], out_hbm.at[pl.ds(pg, GP), pl.ds(c0, CCH)],
                        sy[par]).wait()

                @plsc.parallel_loop(0, GP, 1, unroll=2)
                def px_body(r):
                    for v in range(CV):
                        o = v * L
                        yb[par][r, pl.ds(o, L)] = (
                            ab[par][r, pl.ds(o, L)]
                            * xb[par][r, pl.ds(o, L)]
                            + bb[par][r, pl.ds(o, L)])

                pltpu.async_copy(
                    yb[par], out_hbm.at[pl.ds(pg, GP), pl.ds(c0, CCH)],
                    sy[par])

                @pl.when(g + 2 < NGRP)
                def _():
                    pltpu.async_copy(
                        x_hbm.at[pl.ds(pg + 2 * GP, GP), pl.ds(c0, CCH)],
                        xb[par], sx[par])
                    pltpu.async_copy(
                        a_sh.at[idx_v.at[pl.ds((g + 2) * GP, GP)]],
                        ab[par], sa[par])
                    pltpu.async_copy(
                        b_sh.at[idx_v.at[pl.ds((g + 2) * GP, GP)]],
                        bb[par], sb[par])

            return 0

        lax.fori_loop(0, NGRP // 2, gp_body, 0)
        pltpu.make_async_copy(
            yb0, out_hbm.at[pl.ds(p0 + (NGRP - 2) * GP, GP),
                            pl.ds(c0, CCH)], sy0).wait()
        pltpu.make_async_copy(
            yb1, out_hbm.at[pl.ds(p0 + (NGRP - 1) * GP, GP),
                            pl.ds(c0, CCH)], sy1).wait()

    return _slotfa


@jax.jit
def kernel(x, slot_assign, alphas, betas):
    b, c, h, w = x.shape
    s = alphas.shape[0]
    xt = jnp.transpose(x, (0, 2, 3, 1)).reshape(b * h * w, c)
    slot1 = slot_assign.reshape(b * h * w).astype(jnp.int32)
    ncr = 16
    a2 = alphas.reshape(s * ncr, c // ncr)
    b2 = betas.reshape(s * ncr, c // ncr)
    out2 = _build(b * h * w, c, s)(xt, slot1, a2, b2)
    out = jnp.transpose(out2.reshape(b, h, w, c), (0, 3, 1, 2))
    return out


# speedup vs baseline: 8.4022x; 1.0090x over previous
import functools

import jax
import jax.numpy as jnp
from jax import lax
from jax.experimental import pallas as pl
from jax.experimental.pallas import tpu as pltpu
from jax.experimental.pallas import tpu_sc as plsc

L = 16
NC, NS = 2, 16
NW = NC * NS


@functools.cache
def _build(P, C, S):
    NCR = 16
    NPQ = NW // NCR
    CCH = C // NCR
    PPW = P // NPQ
    GP = 32
    NGRP = PPW // GP
    CV = CCH // L
    mesh = plsc.VectorSubcoreMesh(core_axis_name="c", subcore_axis_name="s")

    @functools.partial(
        pl.kernel,
        mesh=mesh,
        out_type=jax.ShapeDtypeStruct((P, C), jnp.float32),
        scratch_types=[
            pltpu.VMEM((PPW,), jnp.int32),
            pltpu.VMEM((PPW,), jnp.int32),
            pltpu.VMEM_SHARED((S * NCR, CCH), jnp.float32),
            pltpu.VMEM_SHARED((S * NCR, CCH), jnp.float32),
            pltpu.VMEM((GP, CCH), jnp.float32),
            pltpu.VMEM((GP, CCH), jnp.float32),
            pltpu.VMEM((GP, CCH), jnp.float32),
            pltpu.VMEM((GP, CCH), jnp.float32),
            pltpu.VMEM((GP, CCH), jnp.float32),
            pltpu.VMEM((GP, CCH), jnp.float32),
            pltpu.VMEM((GP, CCH), jnp.float32),
            pltpu.VMEM((GP, CCH), jnp.float32),
            pltpu.SemaphoreType.DMA,
            pltpu.SemaphoreType.DMA,
            pltpu.SemaphoreType.DMA,
            pltpu.SemaphoreType.DMA,
            pltpu.SemaphoreType.DMA,
            pltpu.SemaphoreType.DMA,
            pltpu.SemaphoreType.DMA,
            pltpu.SemaphoreType.DMA,
        ],
        compiler_params=pltpu.CompilerParams(needs_layout_passes=False),
    )
    def _slotfa(x_hbm, slot_hbm, a_hbm, b_hbm, out_hbm,
                slot_v, idx_v, a_sh, b_sh,
                xb0, xb1, yb0, yb1, ab0, ab1, bb0, bb1,
                sx0, sx1, sy0, sy1, sa0, sa1, sb0, sb1):
        xb, yb, ab, bb = (xb0, xb1), (yb0, yb1), (ab0, ab1), (bb0, bb1)
        sx, sy, sa, sb = (sx0, sx1), (sy0, sy1), (sa0, sa1), (sb0, sb1)
        cidx = lax.axis_index("c")
        sidx = lax.axis_index("s")
        wid = sidx * NC + cidx
        cr = wid % NCR
        pq = wid // NCR
        c0 = cr * CCH
        p0 = pq * PPW

        @pl.when(sidx == 0)
        def _():
            pltpu.sync_copy(a_hbm, a_sh)
            pltpu.sync_copy(b_hbm, b_sh)

        pltpu.sync_copy(slot_hbm.at[pl.ds(p0, PPW)], slot_v)

        @plsc.parallel_loop(0, PPW // L, 1, unroll=8)
        def idx_body(i):
            o = i * L
            idx_v[pl.ds(o, L)] = slot_v[pl.ds(o, L)] * NCR + cr

        plsc.subcore_barrier()

        for par in range(2):
            pltpu.async_copy(
                x_hbm.at[pl.ds(p0 + par * GP, GP), pl.ds(c0, CCH)],
                xb[par], sx[par])
            pltpu.async_copy(
                a_sh.at[idx_v.at[pl.ds(par * GP, GP)]], ab[par], sa[par])
            pltpu.async_copy(
                b_sh.at[idx_v.at[pl.ds(par * GP, GP)]], bb[par], sb[par])

        def gp_body(gp, _):
            for par in range(2):
                g = gp * 2 + par
                pg = p0 + g * GP
                pltpu.make_async_copy(
                    x_hbm.at[pl.ds(pg, GP), pl.ds(c0, CCH)],
                    xb[par], sx[par]).wait()
                pltpu.make_async_copy(
                    a_sh.at[idx_v.at[pl.ds(g * GP, GP)]], ab[par],
                    sa[par]).wait()
                pltpu.make_async_copy(
                    b_sh.at[idx_v.at[pl.ds(g * GP, GP)]], bb[par],
                    sb[par]).wait()

                @pl.when(gp >= 1)
                def _():
                    pltpu.make_async_copy(
                        yb[par], out_hbm.at[pl.ds(pg, GP), pl.ds(c0, CCH)],
                        sy[par]).wait()

                @plsc.parallel_loop(0, GP, 1, unroll=4)
                def px_body(r):
                    for v in range(CV):
                        o = v * L
                        yb[par][r, pl.ds(o, L)] = (
                            ab[par][r, pl.ds(o, L)]
                            * xb[par][r, pl.ds(o, L)]
                            + bb[par][r, pl.ds(o, L)])

                pltpu.async_copy(
                    yb[par], out_hbm.at[pl.ds(pg, GP), pl.ds(c0, CCH)],
                    sy[par])

                @pl.when(g + 2 < NGRP)
                def _():
                    pltpu.async_copy(
                        x_hbm.at[pl.ds(pg + 2 * GP, GP), pl.ds(c0, CCH)],
                        xb[par], sx[par])
                    pltpu.async_copy(
                        a_sh.at[idx_v.at[pl.ds((g + 2) * GP, GP)]],
                        ab[par], sa[par])
                    pltpu.async_copy(
                        b_sh.at[idx_v.at[pl.ds((g + 2) * GP, GP)]],
                        bb[par], sb[par])

            return 0

        lax.fori_loop(0, NGRP // 2, gp_body, 0)
        pltpu.make_async_copy(
            yb0, out_hbm.at[pl.ds(p0 + (NGRP - 2) * GP, GP),
                            pl.ds(c0, CCH)], sy0).wait()
        pltpu.make_async_copy(
            yb1, out_hbm.at[pl.ds(p0 + (NGRP - 1) * GP, GP),
                            pl.ds(c0, CCH)], sy1).wait()

    return _slotfa


@jax.jit
def kernel(x, slot_assign, alphas, betas):
    b, c, h, w = x.shape
    s = alphas.shape[0]
    xt = jnp.transpose(x, (0, 2, 3, 1)).reshape(b * h * w, c)
    slot1 = slot_assign.reshape(b * h * w).astype(jnp.int32)
    ncr = 16
    a2 = alphas.reshape(s * ncr, c // ncr)
    b2 = betas.reshape(s * ncr, c // ncr)
    out2 = _build(b * h * w, c, s)(xt, slot1, a2, b2)
    out = jnp.transpose(out2.reshape(b, h, w, c), (0, 3, 1, 2))
    return out
